# baseline (speedup 1.0000x reference)
import functools

import jax
import jax.numpy as jnp
from jax.experimental import pallas as pl
from jax.experimental.pallas import tpu as pltpu


def _moe_body(x_ref, wg_ref, bg_ref, w1_ref, b1_ref, w2_ref, b2_ref,
              out_ref, w_scr):
    i_e = pl.program_id(0)
    i_f = pl.program_id(1)
    i_t = pl.program_id(2)
    tm = out_ref.shape[0] // pl.num_programs(2)
    ts = pl.ds(i_t * tm, tm)

    x_t = x_ref[ts, :]

    first_visit = (i_e == 0) & (i_f == 0)

    @pl.when(first_visit)
    def _():
        scores = jnp.dot(x_t, wg_ref[...],
                         preferred_element_type=jnp.float32) + bg_ref[...]
        m = jnp.max(scores, axis=-1, keepdims=True)
        ex = jnp.exp(scores - m)
        w_scr[ts, :] = ex / jnp.sum(ex, axis=-1, keepdims=True)

    h = jnp.dot(x_t, w1_ref[0], preferred_element_type=jnp.float32)
    h = h + b1_ref[0]
    h = 0.5 * h * (1.0 + jax.lax.erf(h * 0.7071067811865476))
    y = jnp.dot(h.astype(jnp.bfloat16), w2_ref[0],
                preferred_element_type=jnp.float32)

    w_tile = w_scr[ts, :]
    lane = jax.lax.broadcasted_iota(jnp.int32, w_tile.shape, 1)
    col = jnp.sum(jnp.where(lane == i_e, w_tile, 0.0), axis=1, keepdims=True)
    contrib = col * y

    @pl.when(first_visit)
    def _():
        out_ref[ts, :] = jnp.dot(w_tile, b2_ref[...],
                                 preferred_element_type=jnp.float32) + contrib

    @pl.when(jnp.logical_not(first_visit))
    def _():
        out_ref[ts, :] += contrib


@functools.partial(jax.jit, static_argnames=())
def kernel(x, Wg, bg, W1, b1, W2, b2):
    T, DIM = x.shape
    E = Wg.shape[1]
    FF = W1.shape[2]
    TM = 512
    FK = 1024
    NT = T // TM
    NF = FF // FK

    bg2 = bg.reshape(1, E)
    b1_3d = b1.reshape(E, 1, FF)
    xb = x.astype(jnp.bfloat16)
    Wgb = Wg.astype(jnp.bfloat16)
    W1b = W1.astype(jnp.bfloat16)
    W2b = W2.astype(jnp.bfloat16)

    return pl.pallas_call(
        _moe_body,
        grid=(E, NF, NT),
        in_specs=[
            pl.BlockSpec((T, DIM), lambda e, f, t: (0, 0)),
            pl.BlockSpec((DIM, E), lambda e, f, t: (0, 0)),
            pl.BlockSpec((1, E), lambda e, f, t: (0, 0)),
            pl.BlockSpec((1, DIM, FK), lambda e, f, t: (e, 0, f)),
            pl.BlockSpec((1, 1, FK), lambda e, f, t: (e, 0, f)),
            pl.BlockSpec((1, FK, DIM), lambda e, f, t: (e, f, 0)),
            pl.BlockSpec((E, DIM), lambda e, f, t: (0, 0)),
        ],
        out_specs=pl.BlockSpec((T, DIM), lambda e, f, t: (0, 0)),
        out_shape=jax.ShapeDtypeStruct((T, DIM), jnp.float32),
        scratch_shapes=[pltpu.VMEM((T, E), jnp.float32)],
        compiler_params=pltpu.CompilerParams(
            dimension_semantics=("arbitrary", "arbitrary", "arbitrary"),
            vmem_limit_bytes=128 * 1024 * 1024,
        ),
    )(xb, Wgb, bg2, W1b, b1_3d, W2b, b2)


# speedup vs baseline: 1.2035x; 1.2035x over previous
import functools

import jax
import jax.numpy as jnp
from jax.experimental import pallas as pl
from jax.experimental.pallas import tpu as pltpu


def _moe_body(x_ref, wg_ref, bg_ref, w1_ref, b1_ref, w2_ref, b2_ref,
              out_ref, w_scr):
    i_e = pl.program_id(0)
    i_f = pl.program_id(1)
    i_t = pl.program_id(2)
    tm = x_ref.shape[0]
    ts = pl.ds(i_t * tm, tm)

    x_t = x_ref[...]

    first_visit = (i_e == 0) & (i_f == 0)

    @pl.when(first_visit)
    def _():
        scores = jnp.dot(x_t, wg_ref[...],
                         preferred_element_type=jnp.float32) + bg_ref[...]
        m = jnp.max(scores, axis=-1, keepdims=True)
        ex = jnp.exp(scores - m)
        w_scr[ts, :] = ex / jnp.sum(ex, axis=-1, keepdims=True)

    h = jnp.dot(x_t, w1_ref[0], preferred_element_type=jnp.float32)
    h = h + b1_ref[0]
    h = 0.5 * h * (1.0 + jax.lax.erf(h * 0.7071067811865476))
    y = jnp.dot(h, w2_ref[0], preferred_element_type=jnp.float32)

    w_tile = w_scr[ts, :]
    lane = jax.lax.broadcasted_iota(jnp.int32, w_tile.shape, 1)
    col = jnp.sum(jnp.where(lane == i_e, w_tile, 0.0), axis=1, keepdims=True)
    contrib = col * y

    @pl.when(first_visit)
    def _():
        out_ref[ts, :] = jnp.dot(w_tile, b2_ref[...],
                                 preferred_element_type=jnp.float32) + contrib

    @pl.when(jnp.logical_not(first_visit))
    def _():
        out_ref[ts, :] += contrib


@functools.partial(jax.jit, static_argnames=())
def kernel(x, Wg, bg, W1, b1, W2, b2):
    T, DIM = x.shape
    E = Wg.shape[1]
    FF = W1.shape[2]
    TM = 1024
    FK = 1024
    NT = T // TM
    NF = FF // FK

    bg2 = bg.reshape(1, E)
    b1_3d = b1.reshape(E, 1, FF)

    return pl.pallas_call(
        _moe_body,
        grid=(E, NF, NT),
        in_specs=[
            pl.BlockSpec((TM, DIM), lambda e, f, t: (t, 0)),
            pl.BlockSpec((DIM, E), lambda e, f, t: (0, 0)),
            pl.BlockSpec((1, E), lambda e, f, t: (0, 0)),
            pl.BlockSpec((1, DIM, FK), lambda e, f, t: (e, 0, f)),
            pl.BlockSpec((1, 1, FK), lambda e, f, t: (e, 0, f)),
            pl.BlockSpec((1, FK, DIM), lambda e, f, t: (e, f, 0)),
            pl.BlockSpec((E, DIM), lambda e, f, t: (0, 0)),
        ],
        out_specs=pl.BlockSpec((T, DIM), lambda e, f, t: (0, 0)),
        out_shape=jax.ShapeDtypeStruct((T, DIM), jnp.float32),
        scratch_shapes=[pltpu.VMEM((T, E), jnp.float32)],
        compiler_params=pltpu.CompilerParams(
            dimension_semantics=("arbitrary", "arbitrary", "arbitrary"),
            vmem_limit_bytes=64 * 1024 * 1024,
        ),
    )(x, Wg, bg2, W1, b1_3d, W2, b2)
